# baseline (speedup 1.0000x reference)
import functools

import jax
import jax.numpy as jnp
from jax import lax
from jax.experimental import pallas as pl
from jax.experimental.pallas import tpu as pltpu
from jax.experimental.pallas import tpu_sc as plsc

F32 = jnp.float32
LATENT = 128
MP = 18
CLOSEST_COUNT = 4
OUT_DIM = 3
N_S, E_S = 10000, 160000
N_D, E_D = 2500, 40000
E_ML = N_D * CLOSEST_COUNT

NCORE, NSUB = 2, 16
NW = NCORE * NSUB
CH = 128

NSP = 10240
NDP = 2560
ESP = 163840
EDP = 40960
EMLP = 12288


def _sc_mesh():
    return plsc.VectorSubcoreMesh(core_axis_name="c", subcore_axis_name="s",
                                  num_cores=NCORE, num_subcores=NSUB)


def _zero_vmem(zbuf, rows, width):
    def zrow(i, c):
        for k in range(width // 16):
            zbuf[i, pl.ds(k * 16, 16)] = jnp.zeros((16,), F32)
        return c
    lax.fori_loop(0, rows, zrow, 0, unroll=False)


@functools.lru_cache(maxsize=None)
def _make_sc_segsum_gather(np_, tablerows, cap_chunks):
    half = np_ // 2
    an = half + CH
    rows_per_sub = an // NSUB
    zrows = max(d for d in range(1, 129) if rows_per_sub % d == 0)
    nz = rows_per_sub // zrows
    out_per_sub = half // NSUB
    D = 4

    @functools.partial(
        pl.kernel, mesh=_sc_mesh(),
        out_type=jax.ShapeDtypeStruct((np_, LATENT), F32),
        scratch_types=[
            pltpu.VMEM((cap_chunks, CH), jnp.int32),
            pltpu.VMEM((cap_chunks, CH), jnp.int32),
            pltpu.VMEM((NSUB, 16), jnp.int32),
            pltpu.VMEM((D, CH, LATENT), F32),
            pltpu.VMEM_SHARED((an, LATENT), F32),
            pltpu.SemaphoreType.DMA,
            pltpu.SemaphoreType.DMA,
        ])
    def k(node_hbm, sidx_hbm, ridx_hbm, cnt_hbm, out_hbm, sidx_v, ridx_v, cnt_v,
          bufv, acc, gsem, ssem):
        cid = lax.axis_index("c")
        sid = lax.axis_index("s")
        _zero_vmem(bufv.at[0], zrows, LATENT)
        def zacc(i, c):
            pltpu.sync_copy(bufv.at[0, pl.ds(0, zrows)],
                            acc.at[pl.ds(sid * rows_per_sub + i * zrows, zrows)])
            return c
        lax.fori_loop(0, nz, zacc, 0, unroll=False)
        pltpu.sync_copy(sidx_hbm.at[cid, sid], sidx_v)
        pltpu.sync_copy(ridx_hbm.at[cid, sid], ridx_v)
        pltpu.sync_copy(cnt_hbm.at[cid], cnt_v)
        plsc.subcore_barrier()
        nw = cnt_v[sid, pl.ds(0, 16)][0]

        def wait_g():
            pltpu.make_async_copy(node_hbm.at[pl.ds(0, CH)], bufv.at[0], gsem).wait()

        def wait_s():
            pltpu.make_async_copy(node_hbm.at[pl.ds(0, CH)], bufv.at[0], ssem).wait()

        K = 2

        def body(j, c):
            jw = j - D
            @pl.when(jnp.logical_and(jw >= 0, jw < nw))
            def _():
                wait_s()
            @pl.when(j < nw)
            def _():
                pltpu.async_copy(node_hbm.at[sidx_v.at[j]], bufv.at[j % D], gsem)
            js = j - K
            @pl.when(jnp.logical_and(js >= 0, js < nw))
            def _():
                wait_g()
                pltpu.async_copy(bufv.at[js % D], acc.at[ridx_v.at[js]], ssem,
                                 add=True)
            return c
        lax.fori_loop(0, nw + D, body, 0, unroll=False)
        plsc.subcore_barrier()
        base = sid * out_per_sub
        pltpu.sync_copy(acc.at[pl.ds(base, out_per_sub)],
                        out_hbm.at[pl.ds(cid * half + base, out_per_sub)])

    return k


@functools.lru_cache(maxsize=None)
def _make_sc_segsum_direct(np_, nchunk, width):
    per_w = nchunk // NW
    rows_per_sub = np_ // NSUB
    zrows = 128 if rows_per_sub % 128 == 0 else rows_per_sub
    nz = rows_per_sub // zrows

    @functools.partial(
        pl.kernel, mesh=_sc_mesh(),
        out_type=jax.ShapeDtypeStruct((NCORE, np_, width), F32),
        scratch_types=[
            pltpu.VMEM((per_w, CH), jnp.int32),
            pltpu.VMEM((CH, width), F32),
            pltpu.VMEM((zrows, width), F32),
            pltpu.VMEM_SHARED((np_, width), F32),
        ])
    def k(vals_hbm, ridx_hbm, out_hbm, ridx_v, buf, zbuf, acc):
        cid = lax.axis_index("c")
        sid = lax.axis_index("s")
        wid = sid * NCORE + cid
        _zero_vmem(zbuf, zrows, width)
        def zacc(i, c):
            pltpu.sync_copy(zbuf, acc.at[pl.ds(sid * rows_per_sub + i * zrows, zrows)])
            return c
        lax.fori_loop(0, nz, zacc, 0, unroll=False)
        pltpu.sync_copy(ridx_hbm.at[wid], ridx_v)
        plsc.subcore_barrier()
        def body(j, c):
            pltpu.sync_copy(vals_hbm.at[pl.ds((wid * per_w + j) * CH, CH)], buf)
            pltpu.sync_copy(buf, acc.at[ridx_v.at[j]], add=True)
            return c
        lax.fori_loop(0, per_w, body, 0, unroll=False)
        plsc.subcore_barrier()
        base = sid * rows_per_sub
        pltpu.sync_copy(acc.at[pl.ds(base, rows_per_sub)],
                        out_hbm.at[cid, pl.ds(base, rows_per_sub)])

    return k


@functools.lru_cache(maxsize=None)
def _make_sc_gather(np_, nchunk):
    per_w = nchunk // NW

    @functools.partial(
        pl.kernel, mesh=_sc_mesh(),
        out_type=jax.ShapeDtypeStruct((nchunk * CH, LATENT), F32),
        scratch_types=[
            pltpu.VMEM((per_w, CH), jnp.int32),
            pltpu.VMEM((CH, LATENT), F32),
            pltpu.SemaphoreType.DMA,
        ])
    def k(table_hbm, idx_hbm, out_hbm, idx_v, buf, sem):
        cid = lax.axis_index("c")
        sid = lax.axis_index("s")
        wid = sid * NCORE + cid
        pltpu.sync_copy(idx_hbm.at[wid], idx_v)
        def body(j, c):
            pltpu.async_copy(table_hbm.at[idx_v.at[j]], buf, sem).wait()
            pltpu.sync_copy(buf, out_hbm.at[pl.ds((wid * per_w + j) * CH, CH)])
            return c
        lax.fori_loop(0, per_w, body, 0, unroll=False)

    return k


def _sc_segsum_gather(node, sidx4d, ridx4d, counts, np_):
    cap = sidx4d.shape[2]
    return _make_sc_segsum_gather(np_, node.shape[0], cap)(node, sidx4d, ridx4d,
                                                           counts)


def _sc_segsum_direct(vals, ridx3d, np_):
    nchunk = ridx3d.shape[0] * ridx3d.shape[1]
    return _make_sc_segsum_direct(np_, nchunk, vals.shape[1])(vals, ridx3d)


def _sc_gather(table, idx3d):
    nchunk = idx3d.shape[0] * idx3d.shape[1]
    return _make_sc_gather(table.shape[0], nchunk)(table, idx3d)


def _dot(a, b):
    return jnp.dot(a, b, preferred_element_type=F32,
                   precision=lax.Precision.HIGHEST)


@functools.lru_cache(maxsize=None)
def _make_tc_encoder(np_, blk):
    grid = np_ // blk

    def body(x_ref, ap_ref, wn_ref, bn_ref, wep_ref, onep_ref, n0_ref, es_ref, dn_ref):
        x = jnp.nan_to_num(x_ref[...])
        n0_ref[...] = _dot(x, wn_ref[...]) + bn_ref[...]
        asum = ap_ref[0] + ap_ref[1]
        es_ref[...] = _dot(asum, wep_ref[...])
        dn_ref[...] = _dot(asum, onep_ref[...])

    return pl.pallas_call(
        body,
        grid=(grid,),
        in_specs=[
            pl.BlockSpec((blk, LATENT), lambda i: (i, 0)),
            pl.BlockSpec((NCORE, blk, LATENT), lambda i: (0, i, 0)),
            pl.BlockSpec((LATENT, LATENT), lambda i: (0, 0)),
            pl.BlockSpec((1, LATENT), lambda i: (0, 0)),
            pl.BlockSpec((LATENT, LATENT), lambda i: (0, 0)),
            pl.BlockSpec((LATENT, LATENT), lambda i: (0, 0)),
        ],
        out_specs=[
            pl.BlockSpec((blk, LATENT), lambda i: (i, 0)),
            pl.BlockSpec((blk, LATENT), lambda i: (i, 0)),
            pl.BlockSpec((blk, LATENT), lambda i: (i, 0)),
        ],
        out_shape=[jax.ShapeDtypeStruct((np_, LATENT), F32)] * 3,
    )


@functools.lru_cache(maxsize=None)
def _make_tc_layer(np_, blk):
    grid = np_ // blk

    def body(n_ref, p_ref, es_ref, dn_ref, wm_ref, bm_ref, wu_ref, bu_ref, out_ref):
        node = n_ref[...]
        dn = dn_ref[...]
        g = p_ref[...]
        segsum = (_dot(g, wm_ref[0:128]) + _dot(es_ref[...], wm_ref[128:256])
                  + _dot(dn * node, wm_ref[256:384]) + dn * bm_ref[...])
        agg = segsum / 10.0
        out_ref[...] = (node + _dot(node, wu_ref[0:128]) + _dot(agg, wu_ref[128:256])
                        + bu_ref[...])

    return pl.pallas_call(
        body,
        grid=(grid,),
        in_specs=[
            pl.BlockSpec((blk, LATENT), lambda i: (i, 0)),
            pl.BlockSpec((blk, LATENT), lambda i: (i, 0)),
            pl.BlockSpec((blk, LATENT), lambda i: (i, 0)),
            pl.BlockSpec((blk, LATENT), lambda i: (i, 0)),
            pl.BlockSpec((3 * LATENT, LATENT), lambda i: (0, 0)),
            pl.BlockSpec((1, LATENT), lambda i: (0, 0)),
            pl.BlockSpec((2 * LATENT, LATENT), lambda i: (0, 0)),
            pl.BlockSpec((1, LATENT), lambda i: (0, 0)),
        ],
        out_specs=pl.BlockSpec((blk, LATENT), lambda i: (i, 0)),
        out_shape=jax.ShapeDtypeStruct((np_, LATENT), F32),
    )


@functools.lru_cache(maxsize=None)
def _make_tc_link_edge(blk):
    grid = E_ML // blk

    def body(gs_ref, ea_ref, gr_ref, wml_ref, bml_ref, wm_ref, bm_ref, out_ref):
        t = _dot(jnp.nan_to_num(ea_ref[...]), wml_ref[...]) + bml_ref[...]
        out_ref[...] = (_dot(gs_ref[...], wm_ref[0:128]) + _dot(t, wm_ref[128:256])
                        + _dot(gr_ref[...], wm_ref[256:384]) + bm_ref[...])

    return pl.pallas_call(
        body,
        grid=(grid,),
        in_specs=[
            pl.BlockSpec((blk, LATENT), lambda i: (i, 0)),
            pl.BlockSpec((blk, 16), lambda i: (i, 0)),
            pl.BlockSpec((blk, LATENT), lambda i: (i, 0)),
            pl.BlockSpec((16, LATENT), lambda i: (0, 0)),
            pl.BlockSpec((1, LATENT), lambda i: (0, 0)),
            pl.BlockSpec((3 * LATENT, LATENT), lambda i: (0, 0)),
            pl.BlockSpec((1, LATENT), lambda i: (0, 0)),
        ],
        out_specs=pl.BlockSpec((blk, LATENT), lambda i: (i, 0)),
        out_shape=jax.ShapeDtypeStruct((E_ML, LATENT), F32),
    )


def _tc_link_reduce(ne2, wu, bu):
    def body(x_ref, w_ref, b_ref, out_ref):
        out_ref[...] = _dot(x_ref[...], w_ref[...]) + b_ref[...]

    return pl.pallas_call(
        body,
        out_shape=jax.ShapeDtypeStruct((N_D, LATENT), F32),
    )(ne2, wu, bu)


def _tc_decoder(nd, wdec, bdec):
    def body(x_ref, w_ref, b_ref, out_ref):
        out_ref[...] = _dot(x_ref[...], w_ref[...]) + b_ref[...]

    return pl.pallas_call(
        body,
        out_shape=jax.ShapeDtypeStruct((nd.shape[0], LATENT), F32),
    )(nd, wdec, bdec)


def _pad_rows(x, n):
    return jnp.pad(x, ((0, n - x.shape[0]),) + ((0, 0),) * (x.ndim - 1))


def _prep_edges_split(ei, ep, np_):
    e = ei.shape[1]
    half = np_ // 2
    order = jnp.argsort(ei[1])
    s_o = ei[0][order]
    r_o = ei[1][order]
    cap = ep // (NSUB * CH)
    count0 = jnp.sum((r_o < half).astype(jnp.int32))
    cnts = jnp.stack([count0, jnp.int32(e) - count0])
    starts = jnp.stack([jnp.int32(0), count0])
    pos = jnp.arange(ep, dtype=jnp.int32)
    gidx = jnp.minimum(pos[None, :] + starts[:, None], e - 1)
    valid = pos[None, :] < cnts[:, None]
    s_c = jnp.where(valid, jnp.take(s_o, gidx), 0)
    r_c = jnp.where(valid, jnp.take(r_o, gidx)
                    - jnp.array([[0], [half]], jnp.int32), half)
    nch = (cnts + CH - 1) // CH
    base = nch // NSUB
    rem = nch % NSUB
    sids = jnp.arange(NSUB, dtype=jnp.int32)
    n_w = base[:, None] + (sids[None, :] < rem[:, None])
    pref = sids[None, :] * base[:, None] + jnp.minimum(sids[None, :],
                                                      rem[:, None])
    k = jnp.arange(cap, dtype=jnp.int32)
    g = pref[:, :, None] + k[None, None, :]
    kvalid = k[None, None, :] < n_w[:, :, None]
    gsafe = jnp.where(kvalid, g, 0)
    lanes = jnp.arange(CH, dtype=jnp.int32)
    epos = gsafe[..., None] * CH + lanes
    take = jax.vmap(lambda arr, idx: jnp.take(arr, idx.reshape(-1)).reshape(idx.shape))
    sidx = jnp.where(kvalid[..., None], take(s_c, epos), 0).astype(jnp.int32)
    ridx = jnp.where(kvalid[..., None], take(r_c, epos), half).astype(jnp.int32)
    counts = jnp.broadcast_to(n_w.astype(jnp.int32)[:, :, None],
                              (NCORE, NSUB, 16))
    r2d = jnp.pad(r_o, (0, ep - e), constant_values=np_ - 1)
    r2d = r2d.reshape(NW, ep // (NW * CH), CH)
    return sidx, ridx, counts, order, r2d


def _attr128(attr, ep):
    e = attr.shape[0]
    a = jnp.nan_to_num(attr)
    a128 = jnp.concatenate([a, jnp.ones((e, 1), F32), jnp.zeros((e, 111), F32)], axis=1)
    return _pad_rows(a128, ep)


def _wepad(we, be):
    return jnp.concatenate([we, be[None, :], jnp.zeros((111, LATENT), F32)], axis=0)


_ONEPAD_ROW = 16


def kernel(sparse_x, sparse_edge_attr, dense_x, dense_edge_attr, multilayer_edge_attr,
           sparse_edge_index, dense_edge_index, multilayer_edge_index,
           We_ns, be_ns, We_es, be_es, We_nd, be_nd, We_ed, be_ed, We_ml, be_ml,
           Wm_s, bm_s, Wu_s, bu_s, Wm_l, bm_l, Wu_l, bu_l,
           Wm_d, bm_d, Wu_d, bu_d, W_dec, b_dec):
    onepad = jnp.zeros((LATENT, LATENT), F32).at[_ONEPAD_ROW].set(1.0)

    si_s, ri_s, cnt_s, order_s, r_s2d = _prep_edges_split(sparse_edge_index, ESP, NSP)
    attr_s = _attr128(sparse_edge_attr[order_s], ESP)
    part_as = _sc_segsum_direct(attr_s, r_s2d, NSP)
    ns, es_const, dn_s = _make_tc_encoder(NSP, 2048)(
        _pad_rows(sparse_x, NSP), part_as, We_ns, be_ns[None, :],
        _wepad(We_es, be_es), onepad)

    layer_s = _make_tc_layer(NSP, 2048)

    def body_s(i, n):
        g = _sc_segsum_gather(n, si_s, ri_s, cnt_s, NSP)
        return layer_s(n, g, es_const, dn_s, Wm_s[i], bm_s[i][None, :],
                       Wu_s[i], bu_s[i][None, :])

    ns = lax.fori_loop(0, MP, body_s, ns)

    si_d, ri_d, cnt_d, order_d, r_d2d = _prep_edges_split(dense_edge_index, EDP, NDP)
    attr_d = _attr128(dense_edge_attr[order_d], EDP)
    part_ad = _sc_segsum_direct(attr_d, r_d2d, NDP)
    nd0, ed_const, dn_d = _make_tc_encoder(NDP, 2560)(
        _pad_rows(dense_x, NDP), part_ad, We_nd, be_nd[None, :],
        _wepad(We_ed, be_ed), onepad)

    sml = jnp.pad(multilayer_edge_index[0], (0, EMLP - E_ML)).reshape(NW, EMLP // (NW * CH), CH)
    rml = jnp.pad(multilayer_edge_index[1], (0, EMLP - E_ML)).reshape(NW, EMLP // (NW * CH), CH)
    gs = _sc_gather(ns, sml)[:E_ML]
    gr = _sc_gather(nd0, rml)[:E_ML]
    ne = _make_tc_link_edge(2000)(gs, multilayer_edge_attr, gr, We_ml, be_ml[None, :],
                                  Wm_l, bm_l[None, :])
    nd = _tc_link_reduce(ne.reshape(N_D, CLOSEST_COUNT * LATENT), Wu_l, bu_l[None, :])
    nd = _pad_rows(nd, NDP)

    layer_d = _make_tc_layer(NDP, 2560)

    def body_d(i, n):
        g = _sc_segsum_gather(n, si_d, ri_d, cnt_d, NDP)
        return layer_d(n, g, ed_const, dn_d, Wm_d[i], bm_d[i][None, :],
                       Wu_d[i], bu_d[i][None, :])

    nd = lax.fori_loop(0, MP, body_d, nd)

    wdec = jnp.pad(W_dec, ((0, 0), (0, LATENT - OUT_DIM)))
    bdec = jnp.pad(b_dec, (0, LATENT - OUT_DIM))[None, :]
    out = _tc_decoder(nd, wdec, bdec)
    return out[:N_D, :OUT_DIM]


# speedup vs baseline: 3.3262x; 3.3262x over previous
import functools

import jax
import jax.numpy as jnp
from jax import lax
from jax.experimental import pallas as pl
from jax.experimental.pallas import tpu as pltpu
from jax.experimental.pallas import tpu_sc as plsc

F32 = jnp.float32
LATENT = 128
MP = 18
CLOSEST_COUNT = 4
OUT_DIM = 3
N_S, E_S = 10000, 160000
N_D, E_D = 2500, 40000
E_ML = N_D * CLOSEST_COUNT

NCORE, NSUB = 2, 16
NW = NCORE * NSUB
CH = 128

NSP = 10240
NDP = 2560
ESP = 163840
EDP = 40960
EMLP = 12288


def _sc_mesh():
    return plsc.VectorSubcoreMesh(core_axis_name="c", subcore_axis_name="s",
                                  num_cores=NCORE, num_subcores=NSUB)


def _zero_vmem(zbuf, rows, width):
    def zrow(i, c):
        for k in range(width // 16):
            zbuf[i, pl.ds(k * 16, 16)] = jnp.zeros((16,), F32)
        return c
    lax.fori_loop(0, rows, zrow, 0, unroll=False)


@functools.lru_cache(maxsize=None)
def _make_sc_segsum_gather(np_, nchunk):
    per_w = nchunk // NW
    rows_per_sub = np_ // NSUB
    zrows = max(d for d in range(1, 129) if rows_per_sub % d == 0)
    nz = rows_per_sub // zrows
    NB = 2
    assert per_w % NB == 0
    ngroups = per_w // NB

    @functools.partial(
        pl.kernel, mesh=_sc_mesh(),
        out_type=jax.ShapeDtypeStruct((NCORE, np_, LATENT), F32),
        scratch_types=[
            pltpu.VMEM((per_w, CH), jnp.int32),
            pltpu.VMEM((per_w, CH), jnp.int32),
            pltpu.VMEM((NB, CH, LATENT), F32),
            pltpu.VMEM_SHARED((np_, LATENT), F32),
            pltpu.SemaphoreType.DMA,
            pltpu.SemaphoreType.DMA,
        ])
    def k(node_hbm, sidx_hbm, ridx_hbm, out_hbm, sidx_v, ridx_v, bufv, acc,
          gsem, ssem):
        bufs = [bufv.at[b] for b in range(NB)]
        cid = lax.axis_index("c")
        sid = lax.axis_index("s")
        wid = sid * NCORE + cid
        _zero_vmem(bufs[0], zrows, LATENT)
        def zacc(i, c):
            pltpu.sync_copy(bufv.at[0, pl.ds(0, zrows)],
                            acc.at[pl.ds(sid * rows_per_sub + i * zrows, zrows)])
            return c
        lax.fori_loop(0, nz, zacc, 0, unroll=False)
        pltpu.sync_copy(sidx_hbm.at[wid], sidx_v)
        pltpu.sync_copy(ridx_hbm.at[wid], ridx_v)
        plsc.subcore_barrier()

        def group(g, c):
            gh = [pltpu.async_copy(node_hbm.at[sidx_v.at[g * NB + b]], bufs[b], gsem)
                  for b in range(NB)]
            sh = []
            for b in range(NB):
                gh[b].wait()
                sh.append(pltpu.async_copy(bufs[b], acc.at[ridx_v.at[g * NB + b]],
                                           ssem, add=True))
            for b in range(NB):
                sh[b].wait()
            return c
        lax.fori_loop(0, ngroups, group, 0, unroll=False)
        plsc.subcore_barrier()
        base = sid * rows_per_sub
        pltpu.sync_copy(acc.at[pl.ds(base, rows_per_sub)],
                        out_hbm.at[cid, pl.ds(base, rows_per_sub)])

    return k


@functools.lru_cache(maxsize=None)
def _make_sc_segsum_direct(np_, nchunk, width):
    per_w = nchunk // NW
    rows_per_sub = np_ // NSUB
    zrows = 128 if rows_per_sub % 128 == 0 else rows_per_sub
    nz = rows_per_sub // zrows

    @functools.partial(
        pl.kernel, mesh=_sc_mesh(),
        out_type=jax.ShapeDtypeStruct((NCORE, np_, width), F32),
        scratch_types=[
            pltpu.VMEM((per_w, CH), jnp.int32),
            pltpu.VMEM((CH, width), F32),
            pltpu.VMEM((zrows, width), F32),
            pltpu.VMEM_SHARED((np_, width), F32),
        ])
    def k(vals_hbm, ridx_hbm, out_hbm, ridx_v, buf, zbuf, acc):
        cid = lax.axis_index("c")
        sid = lax.axis_index("s")
        wid = sid * NCORE + cid
        _zero_vmem(zbuf, zrows, width)
        def zacc(i, c):
            pltpu.sync_copy(zbuf, acc.at[pl.ds(sid * rows_per_sub + i * zrows, zrows)])
            return c
        lax.fori_loop(0, nz, zacc, 0, unroll=False)
        pltpu.sync_copy(ridx_hbm.at[wid], ridx_v)
        plsc.subcore_barrier()
        def body(j, c):
            pltpu.sync_copy(vals_hbm.at[pl.ds((wid * per_w + j) * CH, CH)], buf)
            pltpu.sync_copy(buf, acc.at[ridx_v.at[j]], add=True)
            return c
        lax.fori_loop(0, per_w, body, 0, unroll=False)
        plsc.subcore_barrier()
        base = sid * rows_per_sub
        pltpu.sync_copy(acc.at[pl.ds(base, rows_per_sub)],
                        out_hbm.at[cid, pl.ds(base, rows_per_sub)])

    return k


@functools.lru_cache(maxsize=None)
def _make_sc_gather(np_, nchunk):
    per_w = nchunk // NW

    @functools.partial(
        pl.kernel, mesh=_sc_mesh(),
        out_type=jax.ShapeDtypeStruct((nchunk * CH, LATENT), F32),
        scratch_types=[
            pltpu.VMEM((per_w, CH), jnp.int32),
            pltpu.VMEM((CH, LATENT), F32),
            pltpu.SemaphoreType.DMA,
        ])
    def k(table_hbm, idx_hbm, out_hbm, idx_v, buf, sem):
        cid = lax.axis_index("c")
        sid = lax.axis_index("s")
        wid = sid * NCORE + cid
        pltpu.sync_copy(idx_hbm.at[wid], idx_v)
        def body(j, c):
            pltpu.async_copy(table_hbm.at[idx_v.at[j]], buf, sem).wait()
            pltpu.sync_copy(buf, out_hbm.at[pl.ds((wid * per_w + j) * CH, CH)])
            return c
        lax.fori_loop(0, per_w, body, 0, unroll=False)

    return k


def _sc_segsum_gather(node, sidx3d, ridx3d):
    nchunk = sidx3d.shape[0] * sidx3d.shape[1]
    return _make_sc_segsum_gather(node.shape[0], nchunk)(node, sidx3d, ridx3d)


def _sc_segsum_direct(vals, ridx3d, np_):
    nchunk = ridx3d.shape[0] * ridx3d.shape[1]
    return _make_sc_segsum_direct(np_, nchunk, vals.shape[1])(vals, ridx3d)


def _sc_gather(table, idx3d):
    nchunk = idx3d.shape[0] * idx3d.shape[1]
    return _make_sc_gather(table.shape[0], nchunk)(table, idx3d)


def _dot(a, b):
    return jnp.dot(a, b, preferred_element_type=F32,
                   precision=lax.Precision.HIGHEST)


@functools.lru_cache(maxsize=None)
def _make_tc_encoder(np_, blk):
    grid = np_ // blk

    def body(x_ref, ap_ref, wn_ref, bn_ref, wep_ref, onep_ref, n0_ref, es_ref, dn_ref):
        x = jnp.nan_to_num(x_ref[...])
        n0_ref[...] = _dot(x, wn_ref[...]) + bn_ref[...]
        asum = ap_ref[0] + ap_ref[1]
        es_ref[...] = _dot(asum, wep_ref[...])
        dn_ref[...] = _dot(asum, onep_ref[...])

    return pl.pallas_call(
        body,
        grid=(grid,),
        in_specs=[
            pl.BlockSpec((blk, LATENT), lambda i: (i, 0)),
            pl.BlockSpec((NCORE, blk, LATENT), lambda i: (0, i, 0)),
            pl.BlockSpec((LATENT, LATENT), lambda i: (0, 0)),
            pl.BlockSpec((1, LATENT), lambda i: (0, 0)),
            pl.BlockSpec((LATENT, LATENT), lambda i: (0, 0)),
            pl.BlockSpec((LATENT, LATENT), lambda i: (0, 0)),
        ],
        out_specs=[
            pl.BlockSpec((blk, LATENT), lambda i: (i, 0)),
            pl.BlockSpec((blk, LATENT), lambda i: (i, 0)),
            pl.BlockSpec((blk, LATENT), lambda i: (i, 0)),
        ],
        out_shape=[jax.ShapeDtypeStruct((np_, LATENT), F32)] * 3,
    )


@functools.lru_cache(maxsize=None)
def _make_tc_layer(np_, blk):
    grid = np_ // blk

    def body(n_ref, p_ref, es_ref, dn_ref, wm_ref, bm_ref, wu_ref, bu_ref, out_ref):
        node = n_ref[...]
        dn = dn_ref[...]
        g = p_ref[0] + p_ref[1]
        segsum = (_dot(g, wm_ref[0:128]) + _dot(es_ref[...], wm_ref[128:256])
                  + _dot(dn * node, wm_ref[256:384]) + dn * bm_ref[...])
        agg = segsum / 10.0
        out_ref[...] = (node + _dot(node, wu_ref[0:128]) + _dot(agg, wu_ref[128:256])
                        + bu_ref[...])

    return pl.pallas_call(
        body,
        grid=(grid,),
        in_specs=[
            pl.BlockSpec((blk, LATENT), lambda i: (i, 0)),
            pl.BlockSpec((NCORE, blk, LATENT), lambda i: (0, i, 0)),
            pl.BlockSpec((blk, LATENT), lambda i: (i, 0)),
            pl.BlockSpec((blk, LATENT), lambda i: (i, 0)),
            pl.BlockSpec((3 * LATENT, LATENT), lambda i: (0, 0)),
            pl.BlockSpec((1, LATENT), lambda i: (0, 0)),
            pl.BlockSpec((2 * LATENT, LATENT), lambda i: (0, 0)),
            pl.BlockSpec((1, LATENT), lambda i: (0, 0)),
        ],
        out_specs=pl.BlockSpec((blk, LATENT), lambda i: (i, 0)),
        out_shape=jax.ShapeDtypeStruct((np_, LATENT), F32),
    )


@functools.lru_cache(maxsize=None)
def _make_tc_link_edge(blk):
    grid = E_ML // blk

    def body(gs_ref, ea_ref, gr_ref, wml_ref, bml_ref, wm_ref, bm_ref, out_ref):
        t = _dot(jnp.nan_to_num(ea_ref[...]), wml_ref[...]) + bml_ref[...]
        out_ref[...] = (_dot(gs_ref[...], wm_ref[0:128]) + _dot(t, wm_ref[128:256])
                        + _dot(gr_ref[...], wm_ref[256:384]) + bm_ref[...])

    return pl.pallas_call(
        body,
        grid=(grid,),
        in_specs=[
            pl.BlockSpec((blk, LATENT), lambda i: (i, 0)),
            pl.BlockSpec((blk, 16), lambda i: (i, 0)),
            pl.BlockSpec((blk, LATENT), lambda i: (i, 0)),
            pl.BlockSpec((16, LATENT), lambda i: (0, 0)),
            pl.BlockSpec((1, LATENT), lambda i: (0, 0)),
            pl.BlockSpec((3 * LATENT, LATENT), lambda i: (0, 0)),
            pl.BlockSpec((1, LATENT), lambda i: (0, 0)),
        ],
        out_specs=pl.BlockSpec((blk, LATENT), lambda i: (i, 0)),
        out_shape=jax.ShapeDtypeStruct((E_ML, LATENT), F32),
    )


def _tc_link_reduce(ne2, wu, bu):
    def body(x_ref, w_ref, b_ref, out_ref):
        out_ref[...] = _dot(x_ref[...], w_ref[...]) + b_ref[...]

    return pl.pallas_call(
        body,
        out_shape=jax.ShapeDtypeStruct((N_D, LATENT), F32),
    )(ne2, wu, bu)


def _tc_decoder(nd, wdec, bdec):
    def body(x_ref, w_ref, b_ref, out_ref):
        out_ref[...] = _dot(x_ref[...], w_ref[...]) + b_ref[...]

    return pl.pallas_call(
        body,
        out_shape=jax.ShapeDtypeStruct((nd.shape[0], LATENT), F32),
    )(nd, wdec, bdec)


def _pad_rows(x, n):
    return jnp.pad(x, ((0, n - x.shape[0]),) + ((0, 0),) * (x.ndim - 1))


def _prep_edges(ei, ep, dummy):
    order = jnp.argsort(ei[1])
    s = jnp.pad(ei[0][order], (0, ep - ei.shape[1]))
    r = jnp.pad(ei[1][order], (0, ep - ei.shape[1]), constant_values=dummy)
    per_w = ep // (NW * CH)
    return s.reshape(NW, per_w, CH), r.reshape(NW, per_w, CH), order


def _attr128(attr, ep):
    e = attr.shape[0]
    a = jnp.nan_to_num(attr)
    a128 = jnp.concatenate([a, jnp.ones((e, 1), F32), jnp.zeros((e, 111), F32)], axis=1)
    return _pad_rows(a128, ep)


def _wepad(we, be):
    return jnp.concatenate([we, be[None, :], jnp.zeros((111, LATENT), F32)], axis=0)


_ONEPAD_ROW = 16


def kernel(sparse_x, sparse_edge_attr, dense_x, dense_edge_attr, multilayer_edge_attr,
           sparse_edge_index, dense_edge_index, multilayer_edge_index,
           We_ns, be_ns, We_es, be_es, We_nd, be_nd, We_ed, be_ed, We_ml, be_ml,
           Wm_s, bm_s, Wu_s, bu_s, Wm_l, bm_l, Wu_l, bu_l,
           Wm_d, bm_d, Wu_d, bu_d, W_dec, b_dec):
    onepad = jnp.zeros((LATENT, LATENT), F32).at[_ONEPAD_ROW].set(1.0)

    s_s2d, r_s2d, order_s = _prep_edges(sparse_edge_index, ESP, NSP - 1)
    attr_s = _attr128(sparse_edge_attr[order_s], ESP)
    part_as = _sc_segsum_direct(attr_s, r_s2d, NSP)
    ns, es_const, dn_s = _make_tc_encoder(NSP, 2048)(
        _pad_rows(sparse_x, NSP), part_as, We_ns, be_ns[None, :],
        _wepad(We_es, be_es), onepad)

    layer_s = _make_tc_layer(NSP, 2048)

    def body_s(i, n):
        part = _sc_segsum_gather(n, s_s2d, r_s2d)
        return layer_s(n, part, es_const, dn_s, Wm_s[i], bm_s[i][None, :],
                       Wu_s[i], bu_s[i][None, :])

    ns = lax.fori_loop(0, MP, body_s, ns)

    s_d2d, r_d2d, order_d = _prep_edges(dense_edge_index, EDP, NDP - 1)
    attr_d = _attr128(dense_edge_attr[order_d], EDP)
    part_ad = _sc_segsum_direct(attr_d, r_d2d, NDP)
    nd0, ed_const, dn_d = _make_tc_encoder(NDP, 2560)(
        _pad_rows(dense_x, NDP), part_ad, We_nd, be_nd[None, :],
        _wepad(We_ed, be_ed), onepad)

    sml = jnp.pad(multilayer_edge_index[0], (0, EMLP - E_ML)).reshape(NW, EMLP // (NW * CH), CH)
    rml = jnp.pad(multilayer_edge_index[1], (0, EMLP - E_ML)).reshape(NW, EMLP // (NW * CH), CH)
    gs = _sc_gather(ns, sml)[:E_ML]
    gr = _sc_gather(nd0, rml)[:E_ML]
    ne = _make_tc_link_edge(2000)(gs, multilayer_edge_attr, gr, We_ml, be_ml[None, :],
                                  Wm_l, bm_l[None, :])
    nd = _tc_link_reduce(ne.reshape(N_D, CLOSEST_COUNT * LATENT), Wu_l, bu_l[None, :])
    nd = _pad_rows(nd, NDP)

    layer_d = _make_tc_layer(NDP, 2560)

    def body_d(i, n):
        part = _sc_segsum_gather(n, s_d2d, r_d2d)
        return layer_d(n, part, ed_const, dn_d, Wm_d[i], bm_d[i][None, :],
                       Wu_d[i], bu_d[i][None, :])

    nd = lax.fori_loop(0, MP, body_d, nd)

    wdec = jnp.pad(W_dec, ((0, 0), (0, LATENT - OUT_DIM)))
    bdec = jnp.pad(b_dec, (0, LATENT - OUT_DIM))[None, :]
    out = _tc_decoder(nd, wdec, bdec)
    return out[:N_D, :OUT_DIM]


# speedup vs baseline: 3.5924x; 1.0801x over previous
import functools

import jax
import jax.numpy as jnp
from jax import lax
from jax.experimental import pallas as pl
from jax.experimental.pallas import tpu as pltpu
from jax.experimental.pallas import tpu_sc as plsc

F32 = jnp.float32
LATENT = 128
MP = 18
CLOSEST_COUNT = 4
OUT_DIM = 3
N_S, E_S = 10000, 160000
N_D, E_D = 2500, 40000
E_ML = N_D * CLOSEST_COUNT

NCORE, NSUB = 2, 16
NW = NCORE * NSUB
CH = 128

NSP = 10240
NDP = 2560
ESP = 163840
EDP = 40960
EMLP = 12288


def _sc_mesh():
    return plsc.VectorSubcoreMesh(core_axis_name="c", subcore_axis_name="s",
                                  num_cores=NCORE, num_subcores=NSUB)


def _zero_vmem(zbuf, rows, width):
    def zrow(i, c):
        for k in range(width // 16):
            zbuf[i, pl.ds(k * 16, 16)] = jnp.zeros((16,), F32)
        return c
    lax.fori_loop(0, rows, zrow, 0, unroll=False)


@functools.lru_cache(maxsize=None)
def _make_sc_segsum_gather(np_, nchunk):
    per_w = nchunk // NW
    rows_per_sub = np_ // NSUB
    zrows = max(d for d in range(1, 129) if rows_per_sub % d == 0)
    nz = rows_per_sub // zrows
    NB = 2
    assert per_w % NB == 0
    ngroups = per_w // NB

    @functools.partial(
        pl.kernel, mesh=_sc_mesh(),
        out_type=jax.ShapeDtypeStruct((NCORE, np_, LATENT), F32),
        scratch_types=[
            pltpu.VMEM((per_w, CH), jnp.int32),
            pltpu.VMEM((per_w, CH), jnp.int32),
            pltpu.VMEM((NB, CH, LATENT), F32),
            pltpu.VMEM_SHARED((np_, LATENT), F32),
            pltpu.SemaphoreType.DMA,
            pltpu.SemaphoreType.DMA,
        ])
    def k(node_hbm, sidx_hbm, ridx_hbm, out_hbm, sidx_v, ridx_v, bufv, acc,
          gsem, ssem):
        bufs = [bufv.at[b] for b in range(NB)]
        cid = lax.axis_index("c")
        sid = lax.axis_index("s")
        wid = sid * NCORE + cid
        _zero_vmem(bufs[0], zrows, LATENT)
        def zacc(i, c):
            pltpu.sync_copy(bufv.at[0, pl.ds(0, zrows)],
                            acc.at[pl.ds(sid * rows_per_sub + i * zrows, zrows)])
            return c
        lax.fori_loop(0, nz, zacc, 0, unroll=False)
        pltpu.sync_copy(sidx_hbm.at[wid], sidx_v)
        pltpu.sync_copy(ridx_hbm.at[wid], ridx_v)
        plsc.subcore_barrier()

        def group(g, c):
            gh = [pltpu.async_copy(node_hbm.at[sidx_v.at[g * NB + b]], bufs[b], gsem)
                  for b in range(NB)]
            sh = []
            for b in range(NB):
                gh[b].wait()
                sh.append(pltpu.async_copy(bufs[b], acc.at[ridx_v.at[g * NB + b]],
                                           ssem, add=True))
            for b in range(NB):
                sh[b].wait()
            return c
        lax.fori_loop(0, ngroups, group, 0, unroll=False)
        plsc.subcore_barrier()
        base = sid * rows_per_sub
        pltpu.sync_copy(acc.at[pl.ds(base, rows_per_sub)],
                        out_hbm.at[cid, pl.ds(base, rows_per_sub)])

    return k


@functools.lru_cache(maxsize=None)
def _make_sc_segsum_direct(np_, nchunk, width):
    per_w = nchunk // NW
    rows_per_sub = np_ // NSUB
    zrows = 128 if rows_per_sub % 128 == 0 else rows_per_sub
    nz = rows_per_sub // zrows

    @functools.partial(
        pl.kernel, mesh=_sc_mesh(),
        out_type=jax.ShapeDtypeStruct((NCORE, np_, width), F32),
        scratch_types=[
            pltpu.VMEM((per_w, CH), jnp.int32),
            pltpu.VMEM((CH, width), F32),
            pltpu.VMEM((zrows, width), F32),
            pltpu.VMEM_SHARED((np_, width), F32),
        ])
    def k(vals_hbm, ridx_hbm, out_hbm, ridx_v, buf, zbuf, acc):
        cid = lax.axis_index("c")
        sid = lax.axis_index("s")
        wid = sid * NCORE + cid
        _zero_vmem(zbuf, zrows, width)
        def zacc(i, c):
            pltpu.sync_copy(zbuf, acc.at[pl.ds(sid * rows_per_sub + i * zrows, zrows)])
            return c
        lax.fori_loop(0, nz, zacc, 0, unroll=False)
        pltpu.sync_copy(ridx_hbm.at[wid], ridx_v)
        plsc.subcore_barrier()
        def body(j, c):
            pltpu.sync_copy(vals_hbm.at[pl.ds((wid * per_w + j) * CH, CH)], buf)
            pltpu.sync_copy(buf, acc.at[ridx_v.at[j]], add=True)
            return c
        lax.fori_loop(0, per_w, body, 0, unroll=False)
        plsc.subcore_barrier()
        base = sid * rows_per_sub
        pltpu.sync_copy(acc.at[pl.ds(base, rows_per_sub)],
                        out_hbm.at[cid, pl.ds(base, rows_per_sub)])

    return k


@functools.lru_cache(maxsize=None)
def _make_sc_gather(np_, nchunk):
    per_w = nchunk // NW

    @functools.partial(
        pl.kernel, mesh=_sc_mesh(),
        out_type=jax.ShapeDtypeStruct((nchunk * CH, LATENT), F32),
        scratch_types=[
            pltpu.VMEM((per_w, CH), jnp.int32),
            pltpu.VMEM((CH, LATENT), F32),
            pltpu.SemaphoreType.DMA,
        ])
    def k(table_hbm, idx_hbm, out_hbm, idx_v, buf, sem):
        cid = lax.axis_index("c")
        sid = lax.axis_index("s")
        wid = sid * NCORE + cid
        pltpu.sync_copy(idx_hbm.at[wid], idx_v)
        def body(j, c):
            pltpu.async_copy(table_hbm.at[idx_v.at[j]], buf, sem).wait()
            pltpu.sync_copy(buf, out_hbm.at[pl.ds((wid * per_w + j) * CH, CH)])
            return c
        lax.fori_loop(0, per_w, body, 0, unroll=False)

    return k


def _sc_segsum_gather(node, sidx3d, ridx3d):
    nchunk = sidx3d.shape[0] * sidx3d.shape[1]
    return _make_sc_segsum_gather(node.shape[0], nchunk)(node, sidx3d, ridx3d)


def _sc_segsum_direct(vals, ridx3d, np_):
    nchunk = ridx3d.shape[0] * ridx3d.shape[1]
    return _make_sc_segsum_direct(np_, nchunk, vals.shape[1])(vals, ridx3d)


def _sc_gather(table, idx3d):
    nchunk = idx3d.shape[0] * idx3d.shape[1]
    return _make_sc_gather(table.shape[0], nchunk)(table, idx3d)


def _dot(a, b):
    return jnp.dot(a, b, preferred_element_type=F32)


@functools.lru_cache(maxsize=None)
def _make_tc_encoder(np_, blk):
    grid = np_ // blk

    def body(x_ref, ap_ref, wn_ref, bn_ref, wep_ref, onep_ref, n0_ref, es_ref, dn_ref):
        x = jnp.nan_to_num(x_ref[...])
        n0_ref[...] = _dot(x, wn_ref[...]) + bn_ref[...]
        asum = ap_ref[0] + ap_ref[1]
        es_ref[...] = _dot(asum, wep_ref[...])
        dn_ref[...] = _dot(asum, onep_ref[...])

    return pl.pallas_call(
        body,
        grid=(grid,),
        in_specs=[
            pl.BlockSpec((blk, LATENT), lambda i: (i, 0)),
            pl.BlockSpec((NCORE, blk, LATENT), lambda i: (0, i, 0)),
            pl.BlockSpec((LATENT, LATENT), lambda i: (0, 0)),
            pl.BlockSpec((1, LATENT), lambda i: (0, 0)),
            pl.BlockSpec((LATENT, LATENT), lambda i: (0, 0)),
            pl.BlockSpec((LATENT, LATENT), lambda i: (0, 0)),
        ],
        out_specs=[
            pl.BlockSpec((blk, LATENT), lambda i: (i, 0)),
            pl.BlockSpec((blk, LATENT), lambda i: (i, 0)),
            pl.BlockSpec((blk, LATENT), lambda i: (i, 0)),
        ],
        out_shape=[jax.ShapeDtypeStruct((np_, LATENT), F32)] * 3,
    )


@functools.lru_cache(maxsize=None)
def _make_tc_layer(np_, blk):
    grid = np_ // blk

    def body(n_ref, p_ref, es_ref, dn_ref, wm_ref, bm_ref, wu_ref, bu_ref, out_ref):
        node = n_ref[...]
        dn = dn_ref[...]
        g = p_ref[0] + p_ref[1]
        segsum = (_dot(g, wm_ref[0:128]) + _dot(es_ref[...], wm_ref[128:256])
                  + _dot(dn * node, wm_ref[256:384]) + dn * bm_ref[...])
        agg = segsum / 10.0
        out_ref[...] = (node + _dot(node, wu_ref[0:128]) + _dot(agg, wu_ref[128:256])
                        + bu_ref[...])

    return pl.pallas_call(
        body,
        grid=(grid,),
        in_specs=[
            pl.BlockSpec((blk, LATENT), lambda i: (i, 0)),
            pl.BlockSpec((NCORE, blk, LATENT), lambda i: (0, i, 0)),
            pl.BlockSpec((blk, LATENT), lambda i: (i, 0)),
            pl.BlockSpec((blk, LATENT), lambda i: (i, 0)),
            pl.BlockSpec((3 * LATENT, LATENT), lambda i: (0, 0)),
            pl.BlockSpec((1, LATENT), lambda i: (0, 0)),
            pl.BlockSpec((2 * LATENT, LATENT), lambda i: (0, 0)),
            pl.BlockSpec((1, LATENT), lambda i: (0, 0)),
        ],
        out_specs=pl.BlockSpec((blk, LATENT), lambda i: (i, 0)),
        out_shape=jax.ShapeDtypeStruct((np_, LATENT), F32),
    )


@functools.lru_cache(maxsize=None)
def _make_tc_link_edge(blk):
    grid = E_ML // blk

    def body(gs_ref, ea_ref, gr_ref, wml_ref, bml_ref, wm_ref, bm_ref, out_ref):
        t = _dot(jnp.nan_to_num(ea_ref[...]), wml_ref[...]) + bml_ref[...]
        out_ref[...] = (_dot(gs_ref[...], wm_ref[0:128]) + _dot(t, wm_ref[128:256])
                        + _dot(gr_ref[...], wm_ref[256:384]) + bm_ref[...])

    return pl.pallas_call(
        body,
        grid=(grid,),
        in_specs=[
            pl.BlockSpec((blk, LATENT), lambda i: (i, 0)),
            pl.BlockSpec((blk, 16), lambda i: (i, 0)),
            pl.BlockSpec((blk, LATENT), lambda i: (i, 0)),
            pl.BlockSpec((16, LATENT), lambda i: (0, 0)),
            pl.BlockSpec((1, LATENT), lambda i: (0, 0)),
            pl.BlockSpec((3 * LATENT, LATENT), lambda i: (0, 0)),
            pl.BlockSpec((1, LATENT), lambda i: (0, 0)),
        ],
        out_specs=pl.BlockSpec((blk, LATENT), lambda i: (i, 0)),
        out_shape=jax.ShapeDtypeStruct((E_ML, LATENT), F32),
    )


def _tc_link_reduce(ne2, wu, bu):
    def body(x_ref, w_ref, b_ref, out_ref):
        out_ref[...] = _dot(x_ref[...], w_ref[...]) + b_ref[...]

    return pl.pallas_call(
        body,
        out_shape=jax.ShapeDtypeStruct((N_D, LATENT), F32),
    )(ne2, wu, bu)


def _tc_decoder(nd, wdec, bdec):
    def body(x_ref, w_ref, b_ref, out_ref):
        out_ref[...] = _dot(x_ref[...], w_ref[...]) + b_ref[...]

    return pl.pallas_call(
        body,
        out_shape=jax.ShapeDtypeStruct((nd.shape[0], LATENT), F32),
    )(nd, wdec, bdec)


def _pad_rows(x, n):
    return jnp.pad(x, ((0, n - x.shape[0]),) + ((0, 0),) * (x.ndim - 1))


def _prep_edges(ei, ep, dummy):
    order = jnp.argsort(ei[1])
    s = jnp.pad(ei[0][order], (0, ep - ei.shape[1]))
    r = jnp.pad(ei[1][order], (0, ep - ei.shape[1]), constant_values=dummy)
    per_w = ep // (NW * CH)
    return s.reshape(NW, per_w, CH), r.reshape(NW, per_w, CH), order


def _attr128(attr, ep):
    e = attr.shape[0]
    a = jnp.nan_to_num(attr)
    a128 = jnp.concatenate([a, jnp.ones((e, 1), F32), jnp.zeros((e, 111), F32)], axis=1)
    return _pad_rows(a128, ep)


def _wepad(we, be):
    return jnp.concatenate([we, be[None, :], jnp.zeros((111, LATENT), F32)], axis=0)


_ONEPAD_ROW = 16


def kernel(sparse_x, sparse_edge_attr, dense_x, dense_edge_attr, multilayer_edge_attr,
           sparse_edge_index, dense_edge_index, multilayer_edge_index,
           We_ns, be_ns, We_es, be_es, We_nd, be_nd, We_ed, be_ed, We_ml, be_ml,
           Wm_s, bm_s, Wu_s, bu_s, Wm_l, bm_l, Wu_l, bu_l,
           Wm_d, bm_d, Wu_d, bu_d, W_dec, b_dec):
    onepad = jnp.zeros((LATENT, LATENT), F32).at[_ONEPAD_ROW].set(1.0)

    s_s2d, r_s2d, order_s = _prep_edges(sparse_edge_index, ESP, NSP - 1)
    attr_s = _attr128(sparse_edge_attr[order_s], ESP)
    part_as = _sc_segsum_direct(attr_s, r_s2d, NSP)
    ns, es_const, dn_s = _make_tc_encoder(NSP, 2048)(
        _pad_rows(sparse_x, NSP), part_as, We_ns, be_ns[None, :],
        _wepad(We_es, be_es), onepad)

    layer_s = _make_tc_layer(NSP, 2048)

    def body_s(i, n):
        part = _sc_segsum_gather(n, s_s2d, r_s2d)
        return layer_s(n, part, es_const, dn_s, Wm_s[i], bm_s[i][None, :],
                       Wu_s[i], bu_s[i][None, :])

    ns = lax.fori_loop(0, MP, body_s, ns)

    s_d2d, r_d2d, order_d = _prep_edges(dense_edge_index, EDP, NDP - 1)
    attr_d = _attr128(dense_edge_attr[order_d], EDP)
    part_ad = _sc_segsum_direct(attr_d, r_d2d, NDP)
    nd0, ed_const, dn_d = _make_tc_encoder(NDP, 2560)(
        _pad_rows(dense_x, NDP), part_ad, We_nd, be_nd[None, :],
        _wepad(We_ed, be_ed), onepad)

    sml = jnp.pad(multilayer_edge_index[0], (0, EMLP - E_ML)).reshape(NW, EMLP // (NW * CH), CH)
    rml = jnp.pad(multilayer_edge_index[1], (0, EMLP - E_ML)).reshape(NW, EMLP // (NW * CH), CH)
    gs = _sc_gather(ns, sml)[:E_ML]
    gr = _sc_gather(nd0, rml)[:E_ML]
    ne = _make_tc_link_edge(2000)(gs, multilayer_edge_attr, gr, We_ml, be_ml[None, :],
                                  Wm_l, bm_l[None, :])
    nd = _tc_link_reduce(ne.reshape(N_D, CLOSEST_COUNT * LATENT), Wu_l, bu_l[None, :])
    nd = _pad_rows(nd, NDP)

    layer_d = _make_tc_layer(NDP, 2560)

    def body_d(i, n):
        part = _sc_segsum_gather(n, s_d2d, r_d2d)
        return layer_d(n, part, ed_const, dn_d, Wm_d[i], bm_d[i][None, :],
                       Wu_d[i], bu_d[i][None, :])

    nd = lax.fori_loop(0, MP, body_d, nd)

    wdec = jnp.pad(W_dec, ((0, 0), (0, LATENT - OUT_DIM)))
    bdec = jnp.pad(b_dec, (0, LATENT - OUT_DIM))[None, :]
    out = _tc_decoder(nd, wdec, bdec)
    return out[:N_D, :OUT_DIM]
